# Initial kernel scaffold; baseline (speedup 1.0000x reference)
import jax
import jax.numpy as jnp
from jax.experimental import pallas as pl


def kernel(x, enc_w1, enc_b1, enc_w2, enc_b2, enc_w3, enc_b3, codebook, dec_w1, dec_b1, dec_w2, dec_b2, dec_w3, dec_b3):
    raise NotImplementedError("write your pallas kernel here")


# speedup vs baseline: 1.5561x; 1.5561x over previous
import functools

import jax
import jax.numpy as jnp
from jax import lax
from jax.experimental import pallas as pl
from jax.experimental.pallas import tpu as pltpu

_B, _DIN, _H, _DC, _K = 16384, 512, 256, 32, 1024
_BS = 1024
_NB = _B // _BS


def _vqvae_body(x_ref, ew1, eb1, ew2, eb2, ew3, eb3, cb, cbt,
                dw1, db1, dw2, db2, dw3, db3,
                dec_ref, zq_ref, loss_ref):
    f32 = jnp.float32
    x = x_ref[...]
    h = jnp.maximum(jnp.dot(x, ew1[...], preferred_element_type=f32) + eb1[...], 0.0)
    h = jnp.maximum(jnp.dot(h, ew2[...], preferred_element_type=f32) + eb2[...], 0.0)
    z = jnp.dot(h, ew3[...], preferred_element_type=f32) + eb3[...]

    cbt_v = cbt[...]
    csq = jnp.sum(cbt_v * cbt_v, axis=0, keepdims=True)
    zsq = jnp.sum(z * z, axis=1, keepdims=True)
    d = zsq - 2.0 * jnp.dot(z, cbt_v, preferred_element_type=f32) + csq

    dmin = jnp.min(d, axis=1, keepdims=True)
    iota_k = lax.broadcasted_iota(jnp.int32, (_BS, _K), 1)
    idx = jnp.min(jnp.where(d == dmin, iota_k, _K), axis=1, keepdims=True)
    onehot = (iota_k == idx).astype(f32)
    e = jnp.dot(onehot, cb[...], preferred_element_type=f32)

    zq = z + (e - z)
    zq_ref[...] = zq
    diff = z - e
    loss_ref[...] = jnp.full((1, 8, 128), jnp.sum(diff * diff), dtype=f32)

    g = jnp.maximum(jnp.dot(e, dw1[...], preferred_element_type=f32) + db1[...], 0.0)
    g = jnp.maximum(jnp.dot(g, dw2[...], preferred_element_type=f32) + db2[...], 0.0)
    dec_ref[...] = jnp.dot(g, dw3[...], preferred_element_type=f32) + db3[...]


def _full(shape):
    return pl.BlockSpec(shape, lambda i: (0,) * len(shape))


@jax.jit
def _vqvae_fused(x, enc_w1, enc_b1, enc_w2, enc_b2, enc_w3, enc_b3,
                 codebook, cb_t, dec_w1, dec_b1, dec_w2, dec_b2, dec_w3, dec_b3):
    dec, zq, loss_parts = pl.pallas_call(
        _vqvae_body,
        grid=(_NB,),
        in_specs=[
            pl.BlockSpec((_BS, _DIN), lambda i: (i, 0)),
            _full((_DIN, _H)), _full((1, _H)),
            _full((_H, _H)), _full((1, _H)),
            _full((_H, _DC)), _full((1, _DC)),
            _full((_K, _DC)), _full((_DC, _K)),
            _full((_DC, _H)), _full((1, _H)),
            _full((_H, _H)), _full((1, _H)),
            _full((_H, _DIN)), _full((1, _DIN)),
        ],
        out_specs=[
            pl.BlockSpec((_BS, _DIN), lambda i: (i, 0)),
            pl.BlockSpec((_BS, _DC), lambda i: (i, 0)),
            pl.BlockSpec((1, 8, 128), lambda i: (i, 0, 0)),
        ],
        out_shape=[
            jax.ShapeDtypeStruct((_B, _DIN), jnp.float32),
            jax.ShapeDtypeStruct((_B, _DC), jnp.float32),
            jax.ShapeDtypeStruct((_NB, 8, 128), jnp.float32),
        ],
        compiler_params=pltpu.CompilerParams(
            dimension_semantics=("arbitrary",),
        ),
    )(x, enc_w1, enc_b1, enc_w2, enc_b2, enc_w3, enc_b3, codebook, cb_t,
      dec_w1, dec_b1, dec_w2, dec_b2, dec_w3, dec_b3)
    loss = jnp.sum(loss_parts[:, 0, 0]) / (_B * _DC)
    return dec, zq, loss, loss


def kernel(x, enc_w1, enc_b1, enc_w2, enc_b2, enc_w3, enc_b3, codebook,
           dec_w1, dec_b1, dec_w2, dec_b2, dec_w3, dec_b3):
    return _vqvae_fused(
        x, enc_w1, enc_b1.reshape(1, -1), enc_w2, enc_b2.reshape(1, -1),
        enc_w3, enc_b3.reshape(1, -1), codebook, codebook.T,
        dec_w1, dec_b1.reshape(1, -1), dec_w2, dec_b2.reshape(1, -1),
        dec_w3, dec_b3.reshape(1, -1))


# speedup vs baseline: 1.6328x; 1.0493x over previous
import functools

import jax
import jax.numpy as jnp
from jax import lax
from jax.experimental import pallas as pl
from jax.experimental.pallas import tpu as pltpu

_B, _DIN, _H, _DC, _K = 16384, 512, 256, 32, 1024
_BS = 1024
_NB = _B // _BS


def _vqvae_body(x_ref, ew1, eb1, ew2, eb2, ew3, eb3, cb, cbt,
                dw1, db1, dw2, db2, dw3, db3,
                dec_ref, zq_ref, loss_ref):
    f32 = jnp.float32
    x = x_ref[...]
    h = jnp.maximum(jnp.dot(x, ew1[...], preferred_element_type=f32) + eb1[...], 0.0)
    h = jnp.maximum(jnp.dot(h, ew2[...], preferred_element_type=f32) + eb2[...], 0.0)
    z = jnp.dot(h, ew3[...], preferred_element_type=f32) + eb3[...]

    cbt_v = cbt[...]
    csq = jnp.sum(cbt_v * cbt_v, axis=0, keepdims=True) * 0.25
    d = jnp.dot(z, cbt_v, preferred_element_type=f32) + csq

    dmin = jnp.min(d, axis=1, keepdims=True)
    iota_k = lax.broadcasted_iota(jnp.int32, (_BS, _K), 1).astype(f32)
    idx = jnp.min(jnp.where(d == dmin, iota_k, float(_K)), axis=1, keepdims=True)
    onehot = (iota_k == idx).astype(f32)
    e = jnp.dot(onehot, cb[...], preferred_element_type=f32)

    zq = z + (e - z)
    zq_ref[...] = zq
    diff = z - e
    loss_ref[...] = jnp.full((1, 8, 128), jnp.sum(diff * diff), dtype=f32)

    g = jnp.maximum(jnp.dot(e, dw1[...], preferred_element_type=f32) + db1[...], 0.0)
    g = jnp.maximum(jnp.dot(g, dw2[...], preferred_element_type=f32) + db2[...], 0.0)
    dec_ref[...] = jnp.dot(g, dw3[...], preferred_element_type=f32) + db3[...]


def _full(shape):
    return pl.BlockSpec(shape, lambda i: (0,) * len(shape))


@jax.jit
def _vqvae_fused(x, enc_w1, enc_b1, enc_w2, enc_b2, enc_w3, enc_b3,
                 codebook, cb_t, dec_w1, dec_b1, dec_w2, dec_b2, dec_w3, dec_b3):
    dec, zq, loss_parts = pl.pallas_call(
        _vqvae_body,
        grid=(_NB,),
        in_specs=[
            pl.BlockSpec((_BS, _DIN), lambda i: (i, 0)),
            _full((_DIN, _H)), _full((1, _H)),
            _full((_H, _H)), _full((1, _H)),
            _full((_H, _DC)), _full((1, _DC)),
            _full((_K, _DC)), _full((_DC, _K)),
            _full((_DC, _H)), _full((1, _H)),
            _full((_H, _H)), _full((1, _H)),
            _full((_H, _DIN)), _full((1, _DIN)),
        ],
        out_specs=[
            pl.BlockSpec((_BS, _DIN), lambda i: (i, 0)),
            pl.BlockSpec((_BS, _DC), lambda i: (i, 0)),
            pl.BlockSpec((1, 8, 128), lambda i: (i, 0, 0)),
        ],
        out_shape=[
            jax.ShapeDtypeStruct((_B, _DIN), jnp.float32),
            jax.ShapeDtypeStruct((_B, _DC), jnp.float32),
            jax.ShapeDtypeStruct((_NB, 8, 128), jnp.float32),
        ],
        compiler_params=pltpu.CompilerParams(
            dimension_semantics=("arbitrary",),
        ),
    )(x, enc_w1, enc_b1, enc_w2, enc_b2, enc_w3, enc_b3, codebook, cb_t,
      dec_w1, dec_b1, dec_w2, dec_b2, dec_w3, dec_b3)
    loss = jnp.sum(loss_parts[:, 0, 0]) / (_B * _DC)
    return dec, zq, loss, loss


def kernel(x, enc_w1, enc_b1, enc_w2, enc_b2, enc_w3, enc_b3, codebook,
           dec_w1, dec_b1, dec_w2, dec_b2, dec_w3, dec_b3):
    return _vqvae_fused(
        x, enc_w1, enc_b1.reshape(1, -1), enc_w2, enc_b2.reshape(1, -1),
        enc_w3, enc_b3.reshape(1, -1), codebook, -2.0 * codebook.T,
        dec_w1, dec_b1.reshape(1, -1), dec_w2, dec_b2.reshape(1, -1),
        dec_w3, dec_b3.reshape(1, -1))


# speedup vs baseline: 1.7225x; 1.0549x over previous
import functools

import jax
import jax.numpy as jnp
from jax import lax
from jax.experimental import pallas as pl
from jax.experimental.pallas import tpu as pltpu

_B, _DIN, _H, _DC, _K = 16384, 512, 256, 32, 1024
_BS = 2048
_NB = _B // _BS


def _vqvae_body(x_ref, ew1, eb1, ew2, eb2, ew3, eb3, cb, cbt,
                dw1, db1, dw2, db2, dw3, db3,
                dec_ref, zq_ref, loss_ref):
    f32 = jnp.float32
    x = x_ref[...]
    h = jnp.maximum(jnp.dot(x, ew1[...], preferred_element_type=f32) + eb1[...], 0.0)
    h = jnp.maximum(jnp.dot(h, ew2[...], preferred_element_type=f32) + eb2[...], 0.0)
    z = jnp.dot(h, ew3[...], preferred_element_type=f32) + eb3[...]

    cbt_v = cbt[...]
    csq = jnp.sum(cbt_v * cbt_v, axis=0, keepdims=True) * 0.25
    d = jnp.dot(z, cbt_v, preferred_element_type=f32) + csq

    dmin = jnp.min(d, axis=1, keepdims=True)
    iota_k = lax.broadcasted_iota(jnp.int32, (_BS, _K), 1).astype(f32)
    idx = jnp.min(jnp.where(d == dmin, iota_k, float(_K)), axis=1, keepdims=True)
    onehot = (iota_k == idx).astype(f32)
    e = jnp.dot(onehot, cb[...], preferred_element_type=f32)

    zq = z + (e - z)
    zq_ref[...] = zq
    diff = z - e
    loss_ref[...] = jnp.full((1, 8, 128), jnp.sum(diff * diff), dtype=f32)

    g = jnp.maximum(jnp.dot(e, dw1[...], preferred_element_type=f32) + db1[...], 0.0)
    g = jnp.maximum(jnp.dot(g, dw2[...], preferred_element_type=f32) + db2[...], 0.0)
    dec_ref[...] = jnp.dot(g, dw3[...], preferred_element_type=f32) + db3[...]


def _full(shape):
    return pl.BlockSpec(shape, lambda i: (0,) * len(shape))


@jax.jit
def _vqvae_fused(x, enc_w1, enc_b1, enc_w2, enc_b2, enc_w3, enc_b3,
                 codebook, cb_t, dec_w1, dec_b1, dec_w2, dec_b2, dec_w3, dec_b3):
    dec, zq, loss_parts = pl.pallas_call(
        _vqvae_body,
        grid=(_NB,),
        in_specs=[
            pl.BlockSpec((_BS, _DIN), lambda i: (i, 0)),
            _full((_DIN, _H)), _full((1, _H)),
            _full((_H, _H)), _full((1, _H)),
            _full((_H, _DC)), _full((1, _DC)),
            _full((_K, _DC)), _full((_DC, _K)),
            _full((_DC, _H)), _full((1, _H)),
            _full((_H, _H)), _full((1, _H)),
            _full((_H, _DIN)), _full((1, _DIN)),
        ],
        out_specs=[
            pl.BlockSpec((_BS, _DIN), lambda i: (i, 0)),
            pl.BlockSpec((_BS, _DC), lambda i: (i, 0)),
            pl.BlockSpec((1, 8, 128), lambda i: (i, 0, 0)),
        ],
        out_shape=[
            jax.ShapeDtypeStruct((_B, _DIN), jnp.float32),
            jax.ShapeDtypeStruct((_B, _DC), jnp.float32),
            jax.ShapeDtypeStruct((_NB, 8, 128), jnp.float32),
        ],
        compiler_params=pltpu.CompilerParams(
            dimension_semantics=("arbitrary",),
        ),
    )(x, enc_w1, enc_b1, enc_w2, enc_b2, enc_w3, enc_b3, codebook, cb_t,
      dec_w1, dec_b1, dec_w2, dec_b2, dec_w3, dec_b3)
    loss = jnp.sum(loss_parts[:, 0, 0]) / (_B * _DC)
    return dec, zq, loss, loss


def kernel(x, enc_w1, enc_b1, enc_w2, enc_b2, enc_w3, enc_b3, codebook,
           dec_w1, dec_b1, dec_w2, dec_b2, dec_w3, dec_b3):
    return _vqvae_fused(
        x, enc_w1, enc_b1.reshape(1, -1), enc_w2, enc_b2.reshape(1, -1),
        enc_w3, enc_b3.reshape(1, -1), codebook, -2.0 * codebook.T,
        dec_w1, dec_b1.reshape(1, -1), dec_w2, dec_b2.reshape(1, -1),
        dec_w3, dec_b3.reshape(1, -1))
